# baseline (speedup 1.0000x reference)
import jax
import jax.numpy as jnp
from jax import lax
from jax.experimental import pallas as pl
from jax.experimental.pallas import tpu as pltpu
from jax.experimental.pallas import tpu_sc as plsc

B = 4096
HIST = 200
NUSERS = 1000000
UD = 64
CD = 32
HIDDEN = 256
OUT = 128

NC = 2
NS = 16
NW = NC * NS
BPW = B // NW
C0 = 96
C1 = HIST - C0
L = 16


def _sc_pool_body(hist_hbm, ctab_hbm, c_hbm, hidx_v, crows_a, crows_b,
                  crows_c, crows_d, pooled_v, sem_a, sem_b, sem_c, sem_d):
    wid = lax.axis_index("s") * NC + lax.axis_index("c")
    base = wid * BPW

    pltpu.sync_copy(hist_hbm.at[pl.ds(base * HIST, BPW * HIST)], hidx_v)

    inv = jnp.float32(1.0 / HIST)

    def issue(s, buf, sem):
        pltpu.async_copy(ctab_hbm.at[hidx_v.at[pl.ds(s * HIST, C0)]],
                         buf.at[pl.ds(0, C0)], sem)
        pltpu.async_copy(ctab_hbm.at[hidx_v.at[pl.ds(s * HIST + C0, C1)]],
                         buf.at[pl.ds(C0, C1)], sem)

    def drain(buf, sem):
        pltpu.make_async_copy(ctab_hbm.at[pl.ds(0, C0)],
                              buf.at[pl.ds(0, C0)], sem).wait()
        pltpu.make_async_copy(ctab_hbm.at[pl.ds(0, C1)],
                              buf.at[pl.ds(C0, C1)], sem).wait()

    def accumulate(buf, s):
        zero = jnp.zeros((L,), jnp.float32)
        a00 = a01 = a10 = a11 = zero
        for r in range(0, HIST, 2):
            a00 = a00 + buf[r, 0:16]
            a01 = a01 + buf[r, 16:32]
            a10 = a10 + buf[r + 1, 0:16]
            a11 = a11 + buf[r + 1, 16:32]
        pooled_v[s, 0:16] = (a00 + a10) * inv
        pooled_v[s, 16:32] = (a01 + a11) * inv

    bufs = (crows_a, crows_b, crows_c, crows_d)
    sems = (sem_a, sem_b, sem_c, sem_d)
    nbuf = len(bufs)
    for k in range(nbuf):
        issue(k, bufs[k], sems[k])

    @pl.loop(0, BPW, step=nbuf)
    def _(s):
        for k in range(nbuf):
            drain(bufs[k], sems[k])
            accumulate(bufs[k], s + k)

            @pl.when(s + k + nbuf < BPW)
            def _():
                issue(s + k + nbuf, bufs[k], sems[k])

    pltpu.sync_copy(pooled_v, c_hbm.at[pl.ds(base, BPW)])


@jax.jit
def _sc_cat_pool(hist_flat, cat_table):
    mesh = plsc.VectorSubcoreMesh(core_axis_name="c", subcore_axis_name="s")
    fn = pl.kernel(
        _sc_pool_body,
        out_type=jax.ShapeDtypeStruct((B, CD), jnp.float32),
        mesh=mesh,
        compiler_params=pltpu.CompilerParams(use_tc_tiling_on_sc=False),
        scratch_types=[
            pltpu.VMEM((BPW * HIST,), jnp.int32),
            pltpu.VMEM((HIST, CD), jnp.float32),
            pltpu.VMEM((HIST, CD), jnp.float32),
            pltpu.VMEM((HIST, CD), jnp.float32),
            pltpu.VMEM((HIST, CD), jnp.float32),
            pltpu.VMEM((BPW, CD), jnp.float32),
            pltpu.SemaphoreType.DMA,
            pltpu.SemaphoreType.DMA,
            pltpu.SemaphoreType.DMA,
            pltpu.SemaphoreType.DMA,
        ],
    )
    return fn(hist_flat, cat_table)


GS = 8


def _ugather_body(idx_ref, *refs):
    out_ref = refs[-1]
    tabs = refs[:GS]
    i = pl.program_id(0)
    lane = lax.broadcasted_iota(jnp.int32, (1, 128), 1)
    for j in range(GS):
        m = idx_ref[GS * i + j] % 128
        e = (lane == m).astype(jnp.float32)
        res = lax.dot_general(tabs[j][...], e,
                              (((1,), (1,)), ((), ())),
                              preferred_element_type=jnp.float32)
        out_ref[0, j, :] = res[:, 0]


@jax.jit
def _tc_user_gather(user_id, utab_t):
    def make_map(j):
        return lambda i, idx: (0, idx[GS * i + j] // 128)

    grid_spec = pltpu.PrefetchScalarGridSpec(
        num_scalar_prefetch=1,
        grid=(B // GS,),
        in_specs=[pl.BlockSpec((UD, 128), make_map(j)) for j in range(GS)],
        out_specs=pl.BlockSpec((1, GS, UD), lambda i, idx: (i, 0, 0)),
    )
    out = pl.pallas_call(
        _ugather_body,
        grid_spec=grid_spec,
        out_shape=jax.ShapeDtypeStruct((B // GS, GS, UD), jnp.float32),
    )(user_id, *([utab_t] * GS))
    return out.reshape(B, UD)


def _mlp_body(u_ref, c_ref, w1u_ref, w1c_ref, b1_ref, w2_ref, b2_ref,
              g_ref, bt_ref, o_ref):
    h = jnp.dot(u_ref[...], w1u_ref[...], preferred_element_type=jnp.float32)
    h = h + jnp.dot(c_ref[...], w1c_ref[...], preferred_element_type=jnp.float32)
    h = jnp.maximum(h + b1_ref[...], 0.0)
    h2 = jnp.dot(h, w2_ref[...], preferred_element_type=jnp.float32) + b2_ref[...]
    mean = jnp.mean(h2, axis=-1, keepdims=True)
    cen = h2 - mean
    var = jnp.mean(cen * cen, axis=-1, keepdims=True)
    o_ref[...] = cen * lax.rsqrt(var + 1e-5) * g_ref[...] + bt_ref[...]


BLK = 512


@jax.jit
def _tc_mlp(u, c, W1u, W1c, b1, W2, b2, gamma, beta):
    grid = (B // BLK,)
    return pl.pallas_call(
        _mlp_body,
        grid=grid,
        in_specs=[
            pl.BlockSpec((BLK, UD), lambda i: (i, 0)),
            pl.BlockSpec((BLK, CD), lambda i: (i, 0)),
            pl.BlockSpec((UD, HIDDEN), lambda i: (0, 0)),
            pl.BlockSpec((CD, HIDDEN), lambda i: (0, 0)),
            pl.BlockSpec((1, HIDDEN), lambda i: (0, 0)),
            pl.BlockSpec((HIDDEN, OUT), lambda i: (0, 0)),
            pl.BlockSpec((1, OUT), lambda i: (0, 0)),
            pl.BlockSpec((1, OUT), lambda i: (0, 0)),
            pl.BlockSpec((1, OUT), lambda i: (0, 0)),
        ],
        out_specs=pl.BlockSpec((BLK, OUT), lambda i: (i, 0)),
        out_shape=jax.ShapeDtypeStruct((B, OUT), jnp.float32),
    )(u, c, W1u, W1c, b1, W2, b2, gamma, beta)


def kernel(user_id, category_hist, user_table, cat_table, W1, b1, W2, b2,
           gamma, beta):
    hist_flat = category_hist.reshape(-1)
    u = _tc_user_gather(user_id, user_table.T)
    c = _sc_cat_pool(hist_flat, cat_table)
    return _tc_mlp(u, c, W1[:UD], W1[UD:], b1.reshape(1, -1), W2,
                   b2.reshape(1, -1), gamma.reshape(1, -1),
                   beta.reshape(1, -1))


# speedup vs baseline: 1.4816x; 1.4816x over previous
import jax
import jax.numpy as jnp
from jax import lax
from jax.experimental import pallas as pl
from jax.experimental.pallas import tpu as pltpu
from jax.experimental.pallas import tpu_sc as plsc

B = 4096
HIST = 200
NUSERS = 1000000
UD = 64
CD = 32
HIDDEN = 256
OUT = 128

NC = 2
NS = 16
NW = NC * NS
BPW = B // NW
C0 = 96
C1 = HIST - C0
L = 16


def _sc_pool_body(hist_hbm, ctab_hbm, c_hbm, hidx_v, crows_a, crows_b,
                  crows_c, crows_d, pooled_v, sem_a, sem_b, sem_c, sem_d):
    wid = lax.axis_index("s") * NC + lax.axis_index("c")
    base = wid * BPW

    pltpu.sync_copy(hist_hbm.at[pl.ds(base * HIST, BPW * HIST)], hidx_v)

    inv = jnp.float32(1.0 / HIST)

    def issue(s, buf, sem):
        pltpu.async_copy(ctab_hbm.at[hidx_v.at[pl.ds(s * HIST, C0)]],
                         buf.at[pl.ds(0, C0)], sem)
        pltpu.async_copy(ctab_hbm.at[hidx_v.at[pl.ds(s * HIST + C0, C1)]],
                         buf.at[pl.ds(C0, C1)], sem)

    def drain(buf, sem):
        pltpu.make_async_copy(ctab_hbm.at[pl.ds(0, C0)],
                              buf.at[pl.ds(0, C0)], sem).wait()
        pltpu.make_async_copy(ctab_hbm.at[pl.ds(0, C1)],
                              buf.at[pl.ds(C0, C1)], sem).wait()

    def accumulate(buf, s):
        zero = jnp.zeros((L,), jnp.float32)
        a00 = a01 = a10 = a11 = zero
        for r in range(0, HIST, 2):
            a00 = a00 + buf[r, 0:16]
            a01 = a01 + buf[r, 16:32]
            a10 = a10 + buf[r + 1, 0:16]
            a11 = a11 + buf[r + 1, 16:32]
        pooled_v[s, 0:16] = (a00 + a10) * inv
        pooled_v[s, 16:32] = (a01 + a11) * inv

    bufs = (crows_a, crows_b, crows_c, crows_d)
    sems = (sem_a, sem_b, sem_c, sem_d)
    nbuf = len(bufs)
    for k in range(nbuf):
        issue(k, bufs[k], sems[k])

    @pl.loop(0, BPW, step=nbuf)
    def _(s):
        for k in range(nbuf):
            drain(bufs[k], sems[k])
            accumulate(bufs[k], s + k)

            @pl.when(s + k + nbuf < BPW)
            def _():
                issue(s + k + nbuf, bufs[k], sems[k])

    pltpu.sync_copy(pooled_v, c_hbm.at[pl.ds(base, BPW)])


@jax.jit
def _sc_cat_pool(hist_flat, cat_table):
    mesh = plsc.VectorSubcoreMesh(core_axis_name="c", subcore_axis_name="s")
    fn = pl.kernel(
        _sc_pool_body,
        out_type=jax.ShapeDtypeStruct((B, CD), jnp.float32),
        mesh=mesh,
        compiler_params=pltpu.CompilerParams(use_tc_tiling_on_sc=False),
        scratch_types=[
            pltpu.VMEM((BPW * HIST,), jnp.int32),
            pltpu.VMEM((HIST, CD), jnp.float32),
            pltpu.VMEM((HIST, CD), jnp.float32),
            pltpu.VMEM((HIST, CD), jnp.float32),
            pltpu.VMEM((HIST, CD), jnp.float32),
            pltpu.VMEM((BPW, CD), jnp.float32),
            pltpu.SemaphoreType.DMA,
            pltpu.SemaphoreType.DMA,
            pltpu.SemaphoreType.DMA,
            pltpu.SemaphoreType.DMA,
        ],
    )
    return fn(hist_flat, cat_table)


GS = 8
NBUF = 4


def _ugather_body(idx_ref, tab_ref, out_ref, bufs_ref, sems):
    i = pl.program_id(0)
    nsteps = pl.num_programs(0)

    def issue(g, b):
        @pl.when(g < nsteps)
        def _():
            for j in range(GS):
                uid = idx_ref[g * GS + j]
                start = (uid // 128) * 128
                pltpu.make_async_copy(
                    tab_ref.at[:, pl.ds(start, 128)],
                    bufs_ref.at[b, :, pl.ds(128 * j, 128)],
                    sems.at[b]).start()

    @pl.when(i == 0)
    def _():
        for b in range(NBUF):
            issue(b, b)

    slot = lax.rem(i, NBUF)
    pltpu.make_async_copy(tab_ref.at[:, pl.ds(0, GS * 128)],
                          bufs_ref.at[slot], sems.at[slot]).wait()

    lane = lax.broadcasted_iota(jnp.int32, (1, 128), 1)
    for j in range(GS):
        m = idx_ref[GS * i + j] % 128
        x = bufs_ref[slot, :, 128 * j:128 * (j + 1)]
        e = (lane == m).astype(jnp.float32)
        res = lax.dot_general(x, e,
                              (((1,), (1,)), ((), ())),
                              preferred_element_type=jnp.float32)
        out_ref[0, j, :] = res[:, 0]

    issue(i + NBUF, slot)


@jax.jit
def _tc_user_gather(user_id, utab_t):
    grid_spec = pltpu.PrefetchScalarGridSpec(
        num_scalar_prefetch=1,
        grid=(B // GS,),
        in_specs=[pl.BlockSpec(memory_space=pl.ANY)],
        out_specs=pl.BlockSpec((1, GS, UD), lambda i, idx: (i, 0, 0)),
        scratch_shapes=[
            pltpu.VMEM((NBUF, UD, GS * 128), jnp.float32),
            pltpu.SemaphoreType.DMA((NBUF,)),
        ],
    )
    out = pl.pallas_call(
        _ugather_body,
        grid_spec=grid_spec,
        out_shape=jax.ShapeDtypeStruct((B // GS, GS, UD), jnp.float32),
    )(user_id, utab_t)
    return out.reshape(B, UD)


def _mlp_body(u_ref, c_ref, w1u_ref, w1c_ref, b1_ref, w2_ref, b2_ref,
              g_ref, bt_ref, o_ref):
    h = jnp.dot(u_ref[...], w1u_ref[...], preferred_element_type=jnp.float32)
    h = h + jnp.dot(c_ref[...], w1c_ref[...], preferred_element_type=jnp.float32)
    h = jnp.maximum(h + b1_ref[...], 0.0)
    h2 = jnp.dot(h, w2_ref[...], preferred_element_type=jnp.float32) + b2_ref[...]
    mean = jnp.mean(h2, axis=-1, keepdims=True)
    cen = h2 - mean
    var = jnp.mean(cen * cen, axis=-1, keepdims=True)
    o_ref[...] = cen * lax.rsqrt(var + 1e-5) * g_ref[...] + bt_ref[...]


BLK = 512


@jax.jit
def _tc_mlp(u, c, W1u, W1c, b1, W2, b2, gamma, beta):
    grid = (B // BLK,)
    return pl.pallas_call(
        _mlp_body,
        grid=grid,
        in_specs=[
            pl.BlockSpec((BLK, UD), lambda i: (i, 0)),
            pl.BlockSpec((BLK, CD), lambda i: (i, 0)),
            pl.BlockSpec((UD, HIDDEN), lambda i: (0, 0)),
            pl.BlockSpec((CD, HIDDEN), lambda i: (0, 0)),
            pl.BlockSpec((1, HIDDEN), lambda i: (0, 0)),
            pl.BlockSpec((HIDDEN, OUT), lambda i: (0, 0)),
            pl.BlockSpec((1, OUT), lambda i: (0, 0)),
            pl.BlockSpec((1, OUT), lambda i: (0, 0)),
            pl.BlockSpec((1, OUT), lambda i: (0, 0)),
        ],
        out_specs=pl.BlockSpec((BLK, OUT), lambda i: (i, 0)),
        out_shape=jax.ShapeDtypeStruct((B, OUT), jnp.float32),
    )(u, c, W1u, W1c, b1, W2, b2, gamma, beta)


def kernel(user_id, category_hist, user_table, cat_table, W1, b1, W2, b2,
           gamma, beta):
    hist_flat = category_hist.reshape(-1)
    u = _tc_user_gather(user_id, user_table.T)
    c = _sc_cat_pool(hist_flat, cat_table)
    return _tc_mlp(u, c, W1[:UD], W1[UD:], b1.reshape(1, -1), W2,
                   b2.reshape(1, -1), gamma.reshape(1, -1),
                   beta.reshape(1, -1))


# speedup vs baseline: 1.8947x; 1.2788x over previous
import jax
import jax.numpy as jnp
from jax import lax
from jax.experimental import pallas as pl
from jax.experimental.pallas import tpu as pltpu
from jax.experimental.pallas import tpu_sc as plsc

B = 4096
HIST = 200
NUSERS = 1000000
UD = 64
CD = 32
HIDDEN = 256
OUT = 128

NC = 2
NS = 16
NW = NC * NS
BPW = B // NW
C0 = 96
C1 = HIST - C0
L = 16


def _sc_pool_body(hist_hbm, ctab_hbm, c_hbm, hidx_v, crows_a, crows_b,
                  crows_c, crows_d, pooled_v, sem_a, sem_b, sem_c, sem_d):
    wid = lax.axis_index("s") * NC + lax.axis_index("c")
    base = wid * BPW

    pltpu.sync_copy(hist_hbm.at[pl.ds(base * HIST, BPW * HIST)], hidx_v)

    inv = jnp.float32(1.0 / HIST)

    def issue(s, buf, sem):
        pltpu.async_copy(ctab_hbm.at[hidx_v.at[pl.ds(s * HIST, C0)]],
                         buf.at[pl.ds(0, C0)], sem)
        pltpu.async_copy(ctab_hbm.at[hidx_v.at[pl.ds(s * HIST + C0, C1)]],
                         buf.at[pl.ds(C0, C1)], sem)

    def drain(buf, sem):
        pltpu.make_async_copy(ctab_hbm.at[pl.ds(0, C0)],
                              buf.at[pl.ds(0, C0)], sem).wait()
        pltpu.make_async_copy(ctab_hbm.at[pl.ds(0, C1)],
                              buf.at[pl.ds(C0, C1)], sem).wait()

    def accumulate(buf, s):
        zero = jnp.zeros((L,), jnp.float32)
        a00 = a01 = a10 = a11 = zero
        for r in range(0, HIST, 2):
            a00 = a00 + buf[r, 0:16]
            a01 = a01 + buf[r, 16:32]
            a10 = a10 + buf[r + 1, 0:16]
            a11 = a11 + buf[r + 1, 16:32]
        pooled_v[s, 0:16] = (a00 + a10) * inv
        pooled_v[s, 16:32] = (a01 + a11) * inv

    bufs = (crows_a, crows_b, crows_c, crows_d)
    sems = (sem_a, sem_b, sem_c, sem_d)
    nbuf = len(bufs)
    for k in range(nbuf):
        issue(k, bufs[k], sems[k])

    @pl.loop(0, BPW, step=nbuf)
    def _(s):
        for k in range(nbuf):
            drain(bufs[k], sems[k])
            accumulate(bufs[k], s + k)

            @pl.when(s + k + nbuf < BPW)
            def _():
                issue(s + k + nbuf, bufs[k], sems[k])

    pltpu.sync_copy(pooled_v, c_hbm.at[pl.ds(base, BPW)])


@jax.jit
def _sc_cat_pool(hist_flat, cat_table):
    mesh = plsc.VectorSubcoreMesh(core_axis_name="c", subcore_axis_name="s")
    fn = pl.kernel(
        _sc_pool_body,
        out_type=jax.ShapeDtypeStruct((B, CD), jnp.float32),
        mesh=mesh,
        compiler_params=pltpu.CompilerParams(use_tc_tiling_on_sc=False),
        scratch_types=[
            pltpu.VMEM((BPW * HIST,), jnp.int32),
            pltpu.VMEM((HIST, CD), jnp.float32),
            pltpu.VMEM((HIST, CD), jnp.float32),
            pltpu.VMEM((HIST, CD), jnp.float32),
            pltpu.VMEM((HIST, CD), jnp.float32),
            pltpu.VMEM((BPW, CD), jnp.float32),
            pltpu.SemaphoreType.DMA,
            pltpu.SemaphoreType.DMA,
            pltpu.SemaphoreType.DMA,
            pltpu.SemaphoreType.DMA,
        ],
    )
    return fn(hist_flat, cat_table)


GS = 16
NBUF = 4


def _ugather_body(idx_ref, tab_ref, out_ref, bufs_ref, sems):
    i = pl.program_id(0)
    nsteps = pl.num_programs(0)

    def issue(g, b):
        @pl.when(g < nsteps)
        def _():
            for j in range(GS):
                uid = idx_ref[g * GS + j]
                start = (uid // 128) * 128
                pltpu.make_async_copy(
                    tab_ref.at[:, pl.ds(start, 128)],
                    bufs_ref.at[b, :, pl.ds(128 * j, 128)],
                    sems.at[b]).start()

    @pl.when(i == 0)
    def _():
        for b in range(NBUF):
            issue(b, b)

    slot = lax.rem(i, NBUF)
    pltpu.make_async_copy(tab_ref.at[:, pl.ds(0, GS * 128)],
                          bufs_ref.at[slot], sems.at[slot]).wait()

    lane = lax.broadcasted_iota(jnp.int32, (1, 128), 1)
    for j in range(GS):
        m = idx_ref[GS * i + j] % 128
        x = bufs_ref[slot, :, 128 * j:128 * (j + 1)]
        e = (lane == m).astype(jnp.float32)
        res = lax.dot_general(x, e,
                              (((1,), (1,)), ((), ())),
                              preferred_element_type=jnp.float32)
        out_ref[0, j, :] = res[:, 0]

    issue(i + NBUF, slot)


@jax.jit
def _tc_user_gather(user_id, utab_t):
    grid_spec = pltpu.PrefetchScalarGridSpec(
        num_scalar_prefetch=1,
        grid=(B // GS,),
        in_specs=[pl.BlockSpec(memory_space=pl.ANY)],
        out_specs=pl.BlockSpec((1, GS, UD), lambda i, idx: (i, 0, 0)),
        scratch_shapes=[
            pltpu.VMEM((NBUF, UD, GS * 128), jnp.float32),
            pltpu.SemaphoreType.DMA((NBUF,)),
        ],
    )
    out = pl.pallas_call(
        _ugather_body,
        grid_spec=grid_spec,
        out_shape=jax.ShapeDtypeStruct((B // GS, GS, UD), jnp.float32),
    )(user_id, utab_t)
    return out.reshape(B, UD)


def _mlp_body(u_ref, c_ref, w1u_ref, w1c_ref, b1_ref, w2_ref, b2_ref,
              g_ref, bt_ref, o_ref):
    h = jnp.dot(u_ref[...], w1u_ref[...], preferred_element_type=jnp.float32)
    h = h + jnp.dot(c_ref[...], w1c_ref[...], preferred_element_type=jnp.float32)
    h = jnp.maximum(h + b1_ref[...], 0.0)
    h2 = jnp.dot(h, w2_ref[...], preferred_element_type=jnp.float32) + b2_ref[...]
    mean = jnp.mean(h2, axis=-1, keepdims=True)
    cen = h2 - mean
    var = jnp.mean(cen * cen, axis=-1, keepdims=True)
    o_ref[...] = cen * lax.rsqrt(var + 1e-5) * g_ref[...] + bt_ref[...]


BLK = 512


@jax.jit
def _tc_mlp(u, c, W1u, W1c, b1, W2, b2, gamma, beta):
    grid = (B // BLK,)
    return pl.pallas_call(
        _mlp_body,
        grid=grid,
        in_specs=[
            pl.BlockSpec((BLK, UD), lambda i: (i, 0)),
            pl.BlockSpec((BLK, CD), lambda i: (i, 0)),
            pl.BlockSpec((UD, HIDDEN), lambda i: (0, 0)),
            pl.BlockSpec((CD, HIDDEN), lambda i: (0, 0)),
            pl.BlockSpec((1, HIDDEN), lambda i: (0, 0)),
            pl.BlockSpec((HIDDEN, OUT), lambda i: (0, 0)),
            pl.BlockSpec((1, OUT), lambda i: (0, 0)),
            pl.BlockSpec((1, OUT), lambda i: (0, 0)),
            pl.BlockSpec((1, OUT), lambda i: (0, 0)),
        ],
        out_specs=pl.BlockSpec((BLK, OUT), lambda i: (i, 0)),
        out_shape=jax.ShapeDtypeStruct((B, OUT), jnp.float32),
    )(u, c, W1u, W1c, b1, W2, b2, gamma, beta)


def kernel(user_id, category_hist, user_table, cat_table, W1, b1, W2, b2,
           gamma, beta):
    hist_flat = category_hist.reshape(-1)
    u = _tc_user_gather(user_id, user_table.T)
    c = _sc_cat_pool(hist_flat, cat_table)
    return _tc_mlp(u, c, W1[:UD], W1[UD:], b1.reshape(1, -1), W2,
                   b2.reshape(1, -1), gamma.reshape(1, -1),
                   beta.reshape(1, -1))
